# baseline (speedup 1.0000x reference)
import functools

import jax
import jax.numpy as jnp
from jax import lax
from jax.experimental import pallas as pl
from jax.experimental.pallas import tpu as pltpu
from jax.experimental.pallas import tpu_sc as plsc

BATCH_SIZE = 16384
LANES = 16
NUM_CORES = 2
NUM_SUBCORES = 16
NUM_WORKERS = NUM_CORES * NUM_SUBCORES
B_PER_W = BATCH_SIZE // NUM_WORKERS
G_CHUNK = 128
N_G = B_PER_W // G_CHUNK
N_V = B_PER_W // LANES


def _body(belief_hbm, gt_hbm, idx_hbm, table_hbm, cb_hbm, part_hbm,
          idx_v, lg_v, b_v, gt_v, cb_v, acc_v, sem, sem_in):
    wid = lax.axis_index("s") * NUM_CORES + lax.axis_index("c")
    base = wid * B_PER_W

    cp_b = pltpu.async_copy(belief_hbm.at[pl.ds(base, B_PER_W)], b_v, sem_in)
    cp_t = pltpu.async_copy(gt_hbm.at[pl.ds(base, B_PER_W)], gt_v, sem_in)
    pltpu.sync_copy(idx_hbm.at[wid], idx_v)
    copies = []
    for j in range(N_G):
        copies.append(
            pltpu.async_copy(
                table_hbm.at[idx_v.at[j]],
                lg_v.at[pl.ds(j * G_CHUNK, G_CHUNK)],
                sem,
            )
        )
    cp_b.wait()
    cp_t.wait()

    acc = jnp.zeros((LANES,), jnp.float32)
    for j in range(N_G):
        copies[j].wait()

        def step(i, a, _j=j):
            sl = pl.ds(_j * G_CHUNK + i * LANES, LANES)
            g = lg_v[sl]
            b = b_v[sl]
            t = gt_v[sl]
            cal = 2.0 / (1.0 + jnp.exp(-g))
            cb = jnp.clip(b * cal, 0.0, 1.0)
            cb_v[sl] = cb
            return a + (cb * (1.0 - t) + 0.5 * (1.0 - cb) * t
                        + 0.1 * jnp.abs(cal - 1.0))

        acc = lax.fori_loop(0, G_CHUNK // LANES, step, acc)

    acc_v[...] = acc
    cp_cb = pltpu.async_copy(cb_v, cb_hbm.at[pl.ds(base, B_PER_W)], sem_in)
    cp_pt = pltpu.async_copy(acc_v, part_hbm.at[wid], sem_in)
    cp_cb.wait()
    cp_pt.wait()


@functools.partial(
    pl.kernel,
    out_type=(
        jax.ShapeDtypeStruct((BATCH_SIZE,), jnp.float32),
        jax.ShapeDtypeStruct((NUM_WORKERS, LANES), jnp.float32),
    ),
    mesh=plsc.VectorSubcoreMesh(core_axis_name="c", subcore_axis_name="s"),
    scratch_types=[
        pltpu.VMEM((N_G, G_CHUNK), jnp.int32),
        pltpu.VMEM((B_PER_W,), jnp.float32),
        pltpu.VMEM((B_PER_W,), jnp.float32),
        pltpu.VMEM((B_PER_W,), jnp.float32),
        pltpu.VMEM((B_PER_W,), jnp.float32),
        pltpu.VMEM((LANES,), jnp.float32),
        pltpu.SemaphoreType.DMA,
        pltpu.SemaphoreType.DMA,
    ],
)
def _doxastic_sc(belief_hbm, gt_hbm, idx_hbm, table_hbm, cb_hbm, part_hbm,
                 idx_v, lg_v, b_v, gt_v, cb_v, acc_v, sem, sem_in):
    _body(belief_hbm, gt_hbm, idx_hbm, table_hbm, cb_hbm, part_hbm,
          idx_v, lg_v, b_v, gt_v, cb_v, acc_v, sem, sem_in)


def kernel(belief_strength, ground_truth, agent_ids, calibration_logits):
    idx = agent_ids.astype(jnp.int32).reshape(NUM_WORKERS, N_G, G_CHUNK)
    cb, parts = _doxastic_sc(belief_strength, ground_truth, idx,
                             calibration_logits)
    loss = parts.sum() * (1.0 / BATCH_SIZE)
    return (loss, cb)


# speedup vs baseline: 1.1015x; 1.1015x over previous
import functools

import jax
import jax.numpy as jnp
from jax import lax
from jax.experimental import pallas as pl
from jax.experimental.pallas import tpu as pltpu
from jax.experimental.pallas import tpu_sc as plsc

BATCH_SIZE = 16384
LANES = 16
NUM_SUBCORES = 16
B_PER_W = BATCH_SIZE // NUM_SUBCORES
G_CHUNK = 128
N_G = B_PER_W // G_CHUNK


def _body(belief_hbm, gt_hbm, idx_hbm, table_hbm, cb_hbm, loss_hbm,
          idx_v, lg_v, b_v, gt_v, cb_v, acc_v, red_v, shared, perm_v,
          sem, sem_in):
    sid = lax.axis_index("s")
    base = sid * B_PER_W

    cp_b = pltpu.async_copy(belief_hbm.at[pl.ds(base, B_PER_W)], b_v, sem_in)
    cp_t = pltpu.async_copy(gt_hbm.at[pl.ds(base, B_PER_W)], gt_v, sem_in)
    pltpu.sync_copy(idx_hbm.at[sid], idx_v)
    copies = []
    for j in range(N_G):
        copies.append(
            pltpu.async_copy(
                table_hbm.at[idx_v.at[j]],
                lg_v.at[pl.ds(j * G_CHUNK, G_CHUNK)],
                sem,
            )
        )
    cp_b.wait()
    cp_t.wait()

    acc = jnp.zeros((LANES,), jnp.float32)
    for j in range(N_G):
        copies[j].wait()

        def step(i, a, _j=j):
            sl = pl.ds(_j * G_CHUNK + i * LANES, LANES)
            g = lg_v[sl]
            b = b_v[sl]
            t = gt_v[sl]
            cal = 2.0 / (1.0 + jnp.exp(-g))
            cb = jnp.clip(b * cal, 0.0, 1.0)
            cb_v[sl] = cb
            return a + (cb * (1.0 - t) + 0.5 * (1.0 - cb) * t
                        + 0.1 * jnp.abs(cal - 1.0))

        acc = lax.fori_loop(0, G_CHUNK // LANES, step, acc)

    acc_v[...] = acc
    pltpu.sync_copy(acc_v, shared.at[pl.ds(sid * LANES, LANES)])
    cp_cb = pltpu.async_copy(cb_v, cb_hbm.at[pl.ds(base, B_PER_W)], sem_in)
    plsc.subcore_barrier()

    @pl.when(sid == 0)
    def _reduce():
        pltpu.sync_copy(shared, red_v)
        tot = red_v[pl.ds(0, LANES)]
        for i in range(1, NUM_SUBCORES):
            tot = tot + red_v[pl.ds(i * LANES, LANES)]
        s = tot[0]
        for i in range(1, LANES):
            s = s + tot[i]
        acc_v[...] = jnp.full((LANES,), s * (1.0 / BATCH_SIZE), jnp.float32)
        pltpu.sync_copy(acc_v, loss_hbm)

    cp_cb.wait()


@functools.partial(
    pl.kernel,
    out_type=(
        jax.ShapeDtypeStruct((BATCH_SIZE,), jnp.float32),
        jax.ShapeDtypeStruct((LANES,), jnp.float32),
    ),
    mesh=plsc.VectorSubcoreMesh(core_axis_name="c", subcore_axis_name="s",
                                num_cores=1),
    scratch_types=[
        pltpu.VMEM((N_G, G_CHUNK), jnp.int32),
        pltpu.VMEM((B_PER_W,), jnp.float32),
        pltpu.VMEM((B_PER_W,), jnp.float32),
        pltpu.VMEM((B_PER_W,), jnp.float32),
        pltpu.VMEM((B_PER_W,), jnp.float32),
        pltpu.VMEM((LANES,), jnp.float32),
        pltpu.VMEM((NUM_SUBCORES * LANES,), jnp.float32),
        pltpu.VMEM_SHARED((NUM_SUBCORES * LANES,), jnp.float32),
        pltpu.VMEM((LANES,), jnp.float32),
        pltpu.SemaphoreType.DMA,
        pltpu.SemaphoreType.DMA,
    ],
)
def _doxastic_sc(belief_hbm, gt_hbm, idx_hbm, table_hbm, cb_hbm, loss_hbm,
                 idx_v, lg_v, b_v, gt_v, cb_v, acc_v, red_v, shared, perm_v,
                 sem, sem_in):
    _body(belief_hbm, gt_hbm, idx_hbm, table_hbm, cb_hbm, loss_hbm,
          idx_v, lg_v, b_v, gt_v, cb_v, acc_v, red_v, shared, perm_v,
          sem, sem_in)


def kernel(belief_strength, ground_truth, agent_ids, calibration_logits):
    idx = agent_ids.astype(jnp.int32).reshape(NUM_SUBCORES, N_G, G_CHUNK)
    cb, loss = _doxastic_sc(belief_strength, ground_truth, idx,
                            calibration_logits)
    return (loss[0], cb)


# speedup vs baseline: 1.1971x; 1.0868x over previous
import functools

import jax
import jax.numpy as jnp
from jax import lax
from jax.experimental import pallas as pl
from jax.experimental.pallas import tpu as pltpu
from jax.experimental.pallas import tpu_sc as plsc

BATCH_SIZE = 16384
LANES = 16
NUM_SUBCORES = 16
B_PER_W = BATCH_SIZE // NUM_SUBCORES
G_CHUNK = 128
N_G = B_PER_W // G_CHUNK
N_V = B_PER_W // LANES


def _body(belief_hbm, gt_hbm, idx_hbm, table_hbm, cb_hbm, loss_hbm,
          idx_v, lg_v, b_v, gt_v, cb_v, acc_v, red_v, shared,
          sem, sem_in):
    sid = lax.axis_index("s")
    base = sid * B_PER_W

    pltpu.sync_copy(idx_hbm.at[sid], idx_v)
    copies = []
    for j in range(N_G):
        copies.append(
            pltpu.async_copy(
                table_hbm.at[idx_v.at[j]],
                lg_v.at[pl.ds(j * G_CHUNK, G_CHUNK)],
                sem,
            )
        )
    cp_b = pltpu.async_copy(belief_hbm.at[pl.ds(base, B_PER_W)], b_v, sem_in)
    cp_t = pltpu.async_copy(gt_hbm.at[pl.ds(base, B_PER_W)], gt_v, sem_in)
    cp_b.wait()
    cp_t.wait()
    for c in copies:
        c.wait()

    def step(i, a):
        for u in range(2):
            sl = pl.ds(i * (2 * LANES) + u * LANES, LANES)
            g = lg_v[sl]
            b = b_v[sl]
            t = gt_v[sl]
            cal = 2.0 / (1.0 + jnp.exp(-g))
            cb = jnp.clip(b * cal, 0.0, 1.0)
            cb_v[sl] = cb
            a = a + (cb * (1.0 - 1.5 * t) + 0.5 * t
                     + 0.1 * jnp.abs(cal - 1.0))
        return a

    acc = lax.fori_loop(0, N_V // 2, step, jnp.zeros((LANES,), jnp.float32))

    acc_v[...] = acc
    pltpu.sync_copy(acc_v, shared.at[pl.ds(sid * LANES, LANES)])
    cp_cb = pltpu.async_copy(cb_v, cb_hbm.at[pl.ds(base, B_PER_W)], sem_in)
    plsc.subcore_barrier()

    @pl.when(sid == 0)
    def _reduce():
        pltpu.sync_copy(shared, red_v)
        tot = red_v[pl.ds(0, LANES)]
        for i in range(1, NUM_SUBCORES):
            tot = tot + red_v[pl.ds(i * LANES, LANES)]
        s = tot[0]
        for i in range(1, LANES):
            s = s + tot[i]
        acc_v[...] = jnp.full((LANES,), s * (1.0 / BATCH_SIZE), jnp.float32)
        pltpu.sync_copy(acc_v, loss_hbm)

    cp_cb.wait()


@functools.partial(
    pl.kernel,
    out_type=(
        jax.ShapeDtypeStruct((BATCH_SIZE,), jnp.float32),
        jax.ShapeDtypeStruct((LANES,), jnp.float32),
    ),
    mesh=plsc.VectorSubcoreMesh(core_axis_name="c", subcore_axis_name="s",
                                num_cores=1),
    scratch_types=[
        pltpu.VMEM((N_G, G_CHUNK), jnp.int32),
        pltpu.VMEM((B_PER_W,), jnp.float32),
        pltpu.VMEM((B_PER_W,), jnp.float32),
        pltpu.VMEM((B_PER_W,), jnp.float32),
        pltpu.VMEM((B_PER_W,), jnp.float32),
        pltpu.VMEM((LANES,), jnp.float32),
        pltpu.VMEM((NUM_SUBCORES * LANES,), jnp.float32),
        pltpu.VMEM_SHARED((NUM_SUBCORES * LANES,), jnp.float32),
        pltpu.SemaphoreType.DMA,
        pltpu.SemaphoreType.DMA,
    ],
)
def _doxastic_sc(belief_hbm, gt_hbm, idx_hbm, table_hbm, cb_hbm, loss_hbm,
                 idx_v, lg_v, b_v, gt_v, cb_v, acc_v, red_v, shared,
                 sem, sem_in):
    _body(belief_hbm, gt_hbm, idx_hbm, table_hbm, cb_hbm, loss_hbm,
          idx_v, lg_v, b_v, gt_v, cb_v, acc_v, red_v, shared,
          sem, sem_in)


def kernel(belief_strength, ground_truth, agent_ids, calibration_logits):
    idx = agent_ids.astype(jnp.int32).reshape(NUM_SUBCORES, N_G, G_CHUNK)
    cb, loss = _doxastic_sc(belief_strength, ground_truth, idx,
                            calibration_logits)
    return (loss[0], cb)
